# baseline (speedup 1.0000x reference)
import functools

import jax
import jax.numpy as jnp
from jax import lax
from jax.experimental import pallas as pl
from jax.experimental.pallas import tpu as pltpu
from jax.experimental.pallas import tpu_sc as plsc

_N = 10000
_E = 320000
_D = 128
_NCLS = 40
_NCLS_PAD = 64

_NCORES = 2
_NSUB = 16
_HALF = _N // 2
_CHUNK = 128
_CPT = 160
_STG = 40
_EPAD = _NSUB * _CPT * _CHUNK
_EROWS = _EPAD // _CHUNK
_ACC_ROWS = 5120
_ZROWS = _ACC_ROWS // _NSUB
_NBUF = 4
_BN = 2000


def _make_agg():
    mesh = plsc.VectorSubcoreMesh(core_axis_name="c", subcore_axis_name="s")

    @functools.partial(
        pl.kernel,
        mesh=mesh,
        out_type=jax.ShapeDtypeStruct((_NCORES, _NSUB, _ZROWS, _D), jnp.float32),
        scratch_types=[
            pltpu.VMEM((_STG, _CHUNK), jnp.int32),
            pltpu.VMEM((_STG, _CHUNK), jnp.int32),
            pltpu.VMEM((_NBUF, _CHUNK, _D), jnp.float32),
            pltpu.VMEM_SHARED((_ACC_ROWS, _D), jnp.float32),
            [pltpu.SemaphoreType.DMA] * _NBUF,
            [pltpu.SemaphoreType.DMA] * _NBUF,
        ],
    )
    def agg(h_hbm, src_hbm, dst_hbm, zero_hbm, out_hbm,
            src_v, dst_v, rows_v, acc, gsems, ssems):
        c = lax.axis_index("c")
        s = lax.axis_index("s")

        pltpu.sync_copy(zero_hbm, acc.at[pl.ds(s * _ZROWS, _ZROWS)])
        plsc.subcore_barrier()

        for t in range(_CPT // _STG):
            base = s * _CPT + t * _STG
            pltpu.sync_copy(src_hbm.at[c, pl.ds(base, _STG)], src_v)
            pltpu.sync_copy(dst_hbm.at[c, pl.ds(base, _STG)], dst_v)

            for b in range(_NBUF):
                pltpu.async_copy(h_hbm.at[src_v.at[b]], rows_v.at[b], gsems[b])

            def body(i, carry):
                for b in range(_NBUF):
                    j = i * _NBUF + b
                    pltpu.make_async_copy(
                        h_hbm.at[src_v.at[j]], rows_v.at[b], gsems[b]).wait()
                    pltpu.async_copy(
                        rows_v.at[b], acc.at[dst_v.at[j]], ssems[b], add=True)
                    nj = j + _NBUF

                    @pl.when(nj < _STG)
                    def _():
                        pltpu.make_async_copy(
                            rows_v.at[b], acc.at[dst_v.at[j]], ssems[b]).wait()
                        pltpu.async_copy(
                            h_hbm.at[src_v.at[nj]], rows_v.at[b], gsems[b])
                return carry

            lax.fori_loop(0, _STG // _NBUF, body, 0)
            for b in range(_NBUF):
                pltpu.make_async_copy(
                    rows_v.at[b], acc.at[dst_v.at[0]], ssems[b]).wait()

        plsc.subcore_barrier()

        pltpu.sync_copy(acc.at[pl.ds(s * _ZROWS, _ZROWS)], out_hbm.at[c, s])

    return agg


_agg = _make_agg()


def _mm_first(h, w, b):
    m = w.shape[1]

    def body(h_ref, w_ref, b_ref, o_ref):
        o_ref[...] = jnp.dot(h_ref[...], w_ref[...],
                             preferred_element_type=jnp.float32) + b_ref[...]

    return pl.pallas_call(
        body,
        grid=(_N // _BN,),
        in_specs=[
            pl.BlockSpec((_BN, _D), lambda i: (i, 0)),
            pl.BlockSpec((_D, m), lambda i: (0, 0)),
            pl.BlockSpec((1, m), lambda i: (0, 0)),
        ],
        out_specs=pl.BlockSpec((_BN, m), lambda i: (i, 0)),
        out_shape=jax.ShapeDtypeStruct((_N, m), jnp.float32),
    )(h, w, b.reshape(1, m))


def _mm_pair(p, w, b):
    m = w.shape[1]

    def body(p_ref, w_ref, b_ref, o_ref):
        o_ref[...] = jnp.dot(p_ref[0], w_ref[...],
                             preferred_element_type=jnp.float32) + b_ref[...]

    return pl.pallas_call(
        body,
        grid=(2,),
        in_specs=[
            pl.BlockSpec((1, _HALF, _D), lambda i: (i, 0, 0)),
            pl.BlockSpec((_D, m), lambda i: (0, 0)),
            pl.BlockSpec((1, m), lambda i: (0, 0)),
        ],
        out_specs=pl.BlockSpec((_HALF, m), lambda i: (i, 0)),
        out_shape=jax.ShapeDtypeStruct((_N, m), jnp.float32),
    )(p, w, b.reshape(1, m))


def _loss_kernel(p3, labels):
    lab3 = labels.reshape(2, 1, _HALF)

    def body(p_ref, lab_ref, o_ref):
        i = pl.program_id(0)
        logits = p_ref[0][:, :_NCLS_PAD]
        col = lax.broadcasted_iota(jnp.int32, (_HALF, _NCLS_PAD), 1)
        x = jnp.where(col < _NCLS, logits, jnp.float32(-1e30))
        mx = jnp.max(x, axis=1, keepdims=True)
        lse = mx[:, 0] + jnp.log(jnp.sum(jnp.exp(x - mx), axis=1))
        lab = lab_ref[0, 0, :]
        picked = jnp.sum(
            jnp.where(col == lab[:, None], logits, 0.0), axis=1)
        part = jnp.sum(lse - picked) * jnp.float32(1.0 / _N)

        @pl.when(i == 0)
        def _():
            o_ref[...] = jnp.zeros((1, 1), jnp.float32)

        o_ref[...] += jnp.full((1, 1), 1.0, jnp.float32) * part

    out = pl.pallas_call(
        body,
        grid=(2,),
        in_specs=[
            pl.BlockSpec((1, _HALF, _D), lambda i: (i, 0, 0)),
            pl.BlockSpec((1, 1, _HALF), lambda i: (i, 0, 0)),
        ],
        out_specs=pl.BlockSpec((1, 1), lambda i: (0, 0)),
        out_shape=jax.ShapeDtypeStruct((1, 1), jnp.float32),
    )(p3, lab3)
    return out[0, 0]


def kernel(features, labels, edge_index, W1, b1, W2, b2, W3, b3):
    dst = edge_index[0]
    src = edge_index[1]
    pad = _EPAD - _E
    own0 = dst < _HALF
    trash = _HALF + dst % 120
    padsrc = jnp.zeros((pad,), jnp.int32)
    paddst = _HALF + (jnp.arange(pad, dtype=jnp.int32) % 120)
    src_p = jnp.stack([
        jnp.concatenate([jnp.where(own0, src, 0), padsrc]),
        jnp.concatenate([jnp.where(own0, 0, src), padsrc]),
    ]).reshape(_NCORES, _EROWS, _CHUNK)
    dst_p = jnp.stack([
        jnp.concatenate([jnp.where(own0, dst, trash), paddst]),
        jnp.concatenate([jnp.where(own0, trash, dst - _HALF), paddst]),
    ]).reshape(_NCORES, _EROWS, _CHUNK)
    zeros = jnp.zeros((_ZROWS, _D), jnp.float32)
    w3p = jnp.pad(W3, ((0, 0), (0, _NCLS_PAD - _NCLS)))
    b3p = jnp.pad(b3, (0, _NCLS_PAD - _NCLS))

    h1 = _mm_first(features, W1, b1)
    p1 = _agg(h1, src_p, dst_p, zeros)
    p1 = p1.reshape(_NCORES, _ACC_ROWS, _D)
    h2 = _mm_pair(p1, W2, b2)
    p2 = _agg(h2, src_p, dst_p, zeros)
    p2 = p2.reshape(_NCORES, _ACC_ROWS, _D)
    h3 = _mm_pair(p2, w3p, b3p)
    h3 = jnp.pad(h3, ((0, 0), (0, _D - _NCLS_PAD)))
    p3 = _agg(h3, src_p, dst_p, zeros)
    p3 = p3.reshape(_NCORES, _ACC_ROWS, _D)
    return _loss_kernel(p3, labels)


# speedup vs baseline: 16.8731x; 16.8731x over previous
import functools

import jax
import jax.numpy as jnp
from jax import lax
from jax.experimental import pallas as pl
from jax.experimental.pallas import tpu as pltpu
from jax.experimental.pallas import tpu_sc as plsc

_N = 10000
_E = 320000
_D = 128
_NCLS = 40
_NCLS_PAD = 64

_NCORES = 2
_NSUB = 16
_HALF = _N // 2
_CHUNK = 128
_CPT = 160
_STG = 40
_EPAD = _NSUB * _CPT * _CHUNK
_EROWS = _EPAD // _CHUNK
_ACC_ROWS = 5120
_ZROWS = _ACC_ROWS // _NSUB
_NBUF = 4
_BN = 2000


def _make_agg():
    mesh = plsc.VectorSubcoreMesh(core_axis_name="c", subcore_axis_name="s")

    @functools.partial(
        pl.kernel,
        mesh=mesh,
        out_type=jax.ShapeDtypeStruct((_NCORES, _NSUB, _ZROWS, _D), jnp.float32),
        scratch_types=[
            pltpu.VMEM((_STG, _CHUNK), jnp.int32),
            pltpu.VMEM((_STG, _CHUNK), jnp.int32),
            pltpu.VMEM((_NBUF, _CHUNK, _D), jnp.float32),
            pltpu.VMEM_SHARED((_ACC_ROWS, _D), jnp.float32),
            [pltpu.SemaphoreType.DMA] * _NBUF,
            [pltpu.SemaphoreType.DMA] * _NBUF,
        ],
    )
    def agg(h_hbm, src_hbm, dst_hbm, zero_hbm, out_hbm,
            src_v, dst_v, rows_v, acc, gsems, ssems):
        c = lax.axis_index("c")
        s = lax.axis_index("s")

        pltpu.sync_copy(zero_hbm, acc.at[pl.ds(s * _ZROWS, _ZROWS)])
        plsc.subcore_barrier()

        for t in range(_CPT // _STG):
            base = s * _CPT + t * _STG
            pltpu.sync_copy(src_hbm.at[c, pl.ds(base, _STG)], src_v)
            pltpu.sync_copy(dst_hbm.at[c, pl.ds(base, _STG)], dst_v)

            for b in range(_NBUF):
                pltpu.async_copy(h_hbm.at[src_v.at[b]], rows_v.at[b], gsems[b])

            def body(i, carry):
                for b in range(_NBUF):
                    j = i * _NBUF + b
                    pltpu.make_async_copy(
                        h_hbm.at[src_v.at[j]], rows_v.at[b], gsems[b]).wait()
                    pltpu.async_copy(
                        rows_v.at[b], acc.at[dst_v.at[j]], ssems[b], add=True)
                    nj = j + _NBUF

                    @pl.when(nj < _STG)
                    def _():
                        pltpu.make_async_copy(
                            rows_v.at[b], acc.at[dst_v.at[j]], ssems[b]).wait()
                        pltpu.async_copy(
                            h_hbm.at[src_v.at[nj]], rows_v.at[b], gsems[b])
                return carry

            lax.fori_loop(0, _STG // _NBUF, body, 0)
            for b in range(_NBUF):
                pltpu.make_async_copy(
                    rows_v.at[b], acc.at[dst_v.at[0]], ssems[b]).wait()

        plsc.subcore_barrier()

        pltpu.sync_copy(acc.at[pl.ds(s * _ZROWS, _ZROWS)], out_hbm.at[c, s])

    return agg


_agg = _make_agg()


def _mm_first(h, w, b):
    m = w.shape[1]

    def body(h_ref, w_ref, b_ref, o_ref):
        o_ref[...] = jnp.dot(h_ref[...], w_ref[...],
                             preferred_element_type=jnp.float32) + b_ref[...]

    return pl.pallas_call(
        body,
        grid=(_N // _BN,),
        in_specs=[
            pl.BlockSpec((_BN, _D), lambda i: (i, 0)),
            pl.BlockSpec((_D, m), lambda i: (0, 0)),
            pl.BlockSpec((1, m), lambda i: (0, 0)),
        ],
        out_specs=pl.BlockSpec((_BN, m), lambda i: (i, 0)),
        out_shape=jax.ShapeDtypeStruct((_N, m), jnp.float32),
    )(h, w, b.reshape(1, m))


def _mm_pair(p, w, b):
    m = w.shape[1]

    def body(p_ref, w_ref, b_ref, o_ref):
        o_ref[...] = jnp.dot(p_ref[0], w_ref[...],
                             preferred_element_type=jnp.float32) + b_ref[...]

    return pl.pallas_call(
        body,
        grid=(2,),
        in_specs=[
            pl.BlockSpec((1, _HALF, _D), lambda i: (i, 0, 0)),
            pl.BlockSpec((_D, m), lambda i: (0, 0)),
            pl.BlockSpec((1, m), lambda i: (0, 0)),
        ],
        out_specs=pl.BlockSpec((_HALF, m), lambda i: (i, 0)),
        out_shape=jax.ShapeDtypeStruct((_N, m), jnp.float32),
    )(p, w, b.reshape(1, m))


def _loss_kernel(p3, labels):
    lab3 = labels.reshape(2, 1, _HALF)

    def body(p_ref, lab_ref, o_ref):
        i = pl.program_id(0)
        logits = p_ref[0][:, :_NCLS_PAD]
        col = lax.broadcasted_iota(jnp.int32, (_HALF, _NCLS_PAD), 1)
        x = jnp.where(col < _NCLS, logits, jnp.float32(-1e30))
        mx = jnp.max(x, axis=1, keepdims=True)
        lse = mx[:, 0] + jnp.log(jnp.sum(jnp.exp(x - mx), axis=1))
        lab = lab_ref[0, 0, :]
        picked = jnp.sum(
            jnp.where(col == lab[:, None], logits, 0.0), axis=1)
        part = jnp.sum(lse - picked) * jnp.float32(1.0 / _N)

        @pl.when(i == 0)
        def _():
            o_ref[...] = jnp.zeros((1, 1), jnp.float32)

        o_ref[...] += jnp.full((1, 1), 1.0, jnp.float32) * part

    out = pl.pallas_call(
        body,
        grid=(2,),
        in_specs=[
            pl.BlockSpec((1, _HALF, _D), lambda i: (i, 0, 0)),
            pl.BlockSpec((1, 1, _HALF), lambda i: (i, 0, 0)),
        ],
        out_specs=pl.BlockSpec((1, 1), lambda i: (0, 0)),
        out_shape=jax.ShapeDtypeStruct((1, 1), jnp.float32),
    )(p3, lab3)
    return out[0, 0]


def kernel(features, labels, edge_index, W1, b1, W2, b2, W3, b3):
    dst = edge_index[0]
    src = edge_index[1]
    pad = _EPAD - _E
    own0 = dst < _HALF
    trash = _HALF + dst % 120
    padsrc = jnp.zeros((pad,), jnp.int32)
    paddst = _HALF + (jnp.arange(pad, dtype=jnp.int32) % 120)
    srcf = jnp.concatenate([src, padsrc])
    src_p = jnp.stack([srcf, srcf]).reshape(_NCORES, _EROWS, _CHUNK)
    dst_p = jnp.stack([
        jnp.concatenate([jnp.where(own0, dst, trash), paddst]),
        jnp.concatenate([jnp.where(own0, trash, dst - _HALF), paddst]),
    ]).reshape(_NCORES, _EROWS, _CHUNK)
    zeros = jnp.zeros((_ZROWS, _D), jnp.float32)
    w3p = jnp.pad(W3, ((0, 0), (0, _NCLS_PAD - _NCLS)))
    b3p = jnp.pad(b3, (0, _NCLS_PAD - _NCLS))

    h1 = _mm_first(features, W1, b1)
    p1 = _agg(h1, src_p, dst_p, zeros)
    p1 = p1.reshape(_NCORES, _ACC_ROWS, _D)
    h2 = _mm_pair(p1, W2, b2)
    p2 = _agg(h2, src_p, dst_p, zeros)
    p2 = p2.reshape(_NCORES, _ACC_ROWS, _D)
    h3 = _mm_pair(p2, w3p, b3p)
    h3 = jnp.pad(h3, ((0, 0), (0, _D - _NCLS_PAD)))
    p3 = _agg(h3, src_p, dst_p, zeros)
    p3 = p3.reshape(_NCORES, _ACC_ROWS, _D)
    return _loss_kernel(p3, labels)


# speedup vs baseline: 97.0944x; 5.7544x over previous
import functools

import jax
import jax.numpy as jnp
from jax import lax
from jax.experimental import pallas as pl
from jax.experimental.pallas import tpu as pltpu
from jax.experimental.pallas import tpu_sc as plsc

_N = 10000
_E = 320000
_D = 128
_NCLS = 40
_NCLS_PAD = 64

_NCORES = 2
_NSUB = 16
_CHUNK = 128
_CPT = 160
_STG = 40
_EPAD = _NSUB * _CPT * _CHUNK
_EROWS = _EPAD // _CHUNK
_ACC_ROWS = 10240
_ZROWS = _ACC_ROWS // _NSUB
_NBUF = 4
_BN = 2000


def _make_agg(d2):
    mesh = plsc.VectorSubcoreMesh(core_axis_name="c", subcore_axis_name="s")

    @functools.partial(
        pl.kernel,
        mesh=mesh,
        compiler_params=pltpu.CompilerParams(use_tc_tiling_on_sc=False),
        out_type=jax.ShapeDtypeStruct((_NCORES, _NSUB, _ZROWS, d2), jnp.float32),
        scratch_types=[
            pltpu.VMEM((_STG, _CHUNK), jnp.int32),
            pltpu.VMEM((_STG, _CHUNK), jnp.int32),
            pltpu.VMEM((_NBUF, _CHUNK, d2), jnp.float32),
            pltpu.VMEM_SHARED((_ACC_ROWS, d2), jnp.float32),
            [pltpu.SemaphoreType.DMA] * _NBUF,
            [pltpu.SemaphoreType.DMA] * _NBUF,
        ],
    )
    def agg(h_hbm, src_hbm, dst_hbm, zero_hbm, out_hbm,
            src_v, dst_v, rows_v, acc, gsems, ssems):
        c = lax.axis_index("c")
        s = lax.axis_index("s")

        pltpu.sync_copy(zero_hbm, acc.at[pl.ds(s * _ZROWS, _ZROWS)])
        plsc.subcore_barrier()

        for t in range(_CPT // _STG):
            base = s * _CPT + t * _STG
            pltpu.sync_copy(src_hbm.at[c, pl.ds(base, _STG)], src_v)
            pltpu.sync_copy(dst_hbm.at[pl.ds(base, _STG)], dst_v)

            for b in range(_NBUF):
                pltpu.async_copy(h_hbm.at[src_v.at[b]], rows_v.at[b], gsems[b])

            def body(i, carry):
                for b in range(_NBUF):
                    j = i * _NBUF + b
                    pltpu.make_async_copy(
                        h_hbm.at[src_v.at[j]], rows_v.at[b], gsems[b]).wait()
                    pltpu.async_copy(
                        rows_v.at[b], acc.at[dst_v.at[j]], ssems[b], add=True)
                    nj = j + _NBUF

                    @pl.when(nj < _STG)
                    def _():
                        pltpu.make_async_copy(
                            rows_v.at[b], acc.at[dst_v.at[j]], ssems[b]).wait()
                        pltpu.async_copy(
                            h_hbm.at[src_v.at[nj]], rows_v.at[b], gsems[b])
                return carry

            lax.fori_loop(0, _STG // _NBUF, body, 0)
            for b in range(_NBUF):
                pltpu.make_async_copy(
                    rows_v.at[b], acc.at[dst_v.at[0]], ssems[b]).wait()

        plsc.subcore_barrier()

        pltpu.sync_copy(acc.at[pl.ds(s * _ZROWS, _ZROWS)], out_hbm.at[c, s])

    return agg


_agg64 = _make_agg(_D // 2)
_agg32 = _make_agg(_NCLS_PAD // 2)


def _mm_first(h, w, b):
    m = w.shape[1]
    m2 = m // 2

    def body(h_ref, w_ref, b_ref, o_ref):
        r = jnp.dot(h_ref[...], w_ref[...],
                    preferred_element_type=jnp.float32) + b_ref[...]
        o_ref[0] = r[:, :m2]
        o_ref[1] = r[:, m2:]

    return pl.pallas_call(
        body,
        grid=(_N // _BN,),
        in_specs=[
            pl.BlockSpec((_BN, _D), lambda i: (i, 0)),
            pl.BlockSpec((_D, m), lambda i: (0, 0)),
            pl.BlockSpec((1, m), lambda i: (0, 0)),
        ],
        out_specs=pl.BlockSpec((2, _BN, m2), lambda i: (0, i, 0)),
        out_shape=jax.ShapeDtypeStruct((2, _N, m2), jnp.float32),
    )(h, w, b.reshape(1, m))


def _mm_pair(p, w, b):
    m = w.shape[1]
    m2 = m // 2
    k2 = _D // 2

    def body(p_ref, w_ref, b_ref, o_ref):
        r = (jnp.dot(p_ref[0], w_ref[:k2, :],
                     preferred_element_type=jnp.float32)
             + jnp.dot(p_ref[1], w_ref[k2:, :],
                       preferred_element_type=jnp.float32)
             + b_ref[...])
        o_ref[0] = r[:, :m2]
        o_ref[1] = r[:, m2:]

    return pl.pallas_call(
        body,
        grid=(_N // _BN,),
        in_specs=[
            pl.BlockSpec((2, _BN, k2), lambda i: (0, i, 0)),
            pl.BlockSpec((_D, m), lambda i: (0, 0)),
            pl.BlockSpec((1, m), lambda i: (0, 0)),
        ],
        out_specs=pl.BlockSpec((2, _BN, m2), lambda i: (0, i, 0)),
        out_shape=jax.ShapeDtypeStruct((2, _N, m2), jnp.float32),
    )(p, w, b.reshape(1, m))


def _loss_kernel(p3, labels):
    lab3 = labels.reshape(_N // _BN, 1, _BN)
    c2 = _NCLS_PAD // 2

    def body(p_ref, lab_ref, o_ref):
        i = pl.program_id(0)
        logits = jnp.concatenate([p_ref[0], p_ref[1]], axis=1)
        col = lax.broadcasted_iota(jnp.int32, (_BN, _NCLS_PAD), 1)
        x = jnp.where(col < _NCLS, logits, jnp.float32(-1e30))
        mx = jnp.max(x, axis=1, keepdims=True)
        lse = mx[:, 0] + jnp.log(jnp.sum(jnp.exp(x - mx), axis=1))
        lab = lab_ref[0, 0, :]
        picked = jnp.sum(
            jnp.where(col == lab[:, None], logits, 0.0), axis=1)
        part = jnp.sum(lse - picked) * jnp.float32(1.0 / _N)

        @pl.when(i == 0)
        def _():
            o_ref[...] = jnp.zeros((1, 1), jnp.float32)

        o_ref[...] += jnp.full((1, 1), 1.0, jnp.float32) * part

    out = pl.pallas_call(
        body,
        grid=(_N // _BN,),
        in_specs=[
            pl.BlockSpec((2, _BN, c2), lambda i: (0, i, 0)),
            pl.BlockSpec((1, 1, _BN), lambda i: (i, 0, 0)),
        ],
        out_specs=pl.BlockSpec((1, 1), lambda i: (0, 0)),
        out_shape=jax.ShapeDtypeStruct((1, 1), jnp.float32),
    )(p3, lab3)
    return out[0, 0]


def kernel(features, labels, edge_index, W1, b1, W2, b2, W3, b3):
    dst = edge_index[0]
    src = edge_index[1]
    pad = _EPAD - _E
    padsrc = jnp.arange(pad, dtype=jnp.int32) % _N
    paddst = _N + (jnp.arange(pad, dtype=jnp.int32) % 120)
    srcf = jnp.concatenate([src, padsrc])
    src_p = jnp.stack([srcf, srcf + _N]).reshape(_NCORES, _EROWS, _CHUNK)
    dst_p = jnp.concatenate([dst, paddst]).reshape(_EROWS, _CHUNK)
    z64 = jnp.zeros((_ZROWS, _D // 2), jnp.float32)
    z32 = jnp.zeros((_ZROWS, _NCLS_PAD // 2), jnp.float32)
    w3p = jnp.pad(W3, ((0, 0), (0, _NCLS_PAD - _NCLS)))
    b3p = jnp.pad(b3, (0, _NCLS_PAD - _NCLS))

    h1 = _mm_first(features, W1, b1).reshape(2 * _N, _D // 2)
    p1 = _agg64(h1, src_p, dst_p, z64)
    p1 = p1.reshape(_NCORES, _ACC_ROWS, _D // 2)
    h2 = _mm_pair(p1, W2, b2).reshape(2 * _N, _D // 2)
    p2 = _agg64(h2, src_p, dst_p, z64)
    p2 = p2.reshape(_NCORES, _ACC_ROWS, _D // 2)
    h3 = _mm_pair(p2, w3p, b3p).reshape(2 * _N, _NCLS_PAD // 2)
    p3 = _agg32(h3, src_p, dst_p, z32)
    p3 = p3.reshape(_NCORES, _ACC_ROWS, _NCLS_PAD // 2)
    return _loss_kernel(p3, labels)


# speedup vs baseline: 119.1508x; 1.2272x over previous
import functools

import jax
import jax.numpy as jnp
from jax import lax
from jax.experimental import pallas as pl
from jax.experimental.pallas import tpu as pltpu
from jax.experimental.pallas import tpu_sc as plsc

_N = 10000
_E = 320000
_D = 128
_NCLS = 40
_NCLS_PAD = 64

_NCORES = 2
_NSUB = 16
_CHUNK = 128
_CPT = 160
_STG = 40
_EPAD = _NSUB * _CPT * _CHUNK
_EROWS = _EPAD // _CHUNK
_ACC_ROWS = 10240
_ZROWS = _ACC_ROWS // _NSUB
_NBUF = 4
_BN = 2000


def _make_agg(d2):
    mesh = plsc.VectorSubcoreMesh(core_axis_name="c", subcore_axis_name="s")

    @functools.partial(
        pl.kernel,
        mesh=mesh,
        compiler_params=pltpu.CompilerParams(use_tc_tiling_on_sc=False),
        out_type=jax.ShapeDtypeStruct((_NCORES, _NSUB, _ZROWS, d2), jnp.bfloat16),
        scratch_types=[
            pltpu.VMEM((_STG, _CHUNK), jnp.int32),
            pltpu.VMEM((_STG, _CHUNK), jnp.int32),
            pltpu.VMEM((_NBUF, _CHUNK, d2), jnp.bfloat16),
            pltpu.VMEM_SHARED((_ACC_ROWS, d2), jnp.bfloat16),
            [pltpu.SemaphoreType.DMA] * _NBUF,
            [pltpu.SemaphoreType.DMA] * _NBUF,
        ],
    )
    def agg(h_hbm, src_hbm, dst_hbm, zero_hbm, out_hbm,
            src_v, dst_v, rows_v, acc, gsems, ssems):
        c = lax.axis_index("c")
        s = lax.axis_index("s")

        pltpu.sync_copy(zero_hbm, acc.at[pl.ds(s * _ZROWS, _ZROWS)])
        plsc.subcore_barrier()

        for t in range(_CPT // _STG):
            base = s * _CPT + t * _STG
            pltpu.sync_copy(src_hbm.at[c, pl.ds(base, _STG)], src_v)
            pltpu.sync_copy(dst_hbm.at[pl.ds(base, _STG)], dst_v)

            for b in range(_NBUF):
                pltpu.async_copy(h_hbm.at[src_v.at[b]], rows_v.at[b], gsems[b])

            def body(i, carry):
                for b in range(_NBUF):
                    j = i * _NBUF + b
                    pltpu.make_async_copy(
                        h_hbm.at[src_v.at[j]], rows_v.at[b], gsems[b]).wait()
                    pltpu.async_copy(
                        rows_v.at[b], acc.at[dst_v.at[j]], ssems[b], add=True)
                    nj = j + _NBUF

                    @pl.when(nj < _STG)
                    def _():
                        pltpu.make_async_copy(
                            rows_v.at[b], acc.at[dst_v.at[j]], ssems[b]).wait()
                        pltpu.async_copy(
                            h_hbm.at[src_v.at[nj]], rows_v.at[b], gsems[b])
                return carry

            lax.fori_loop(0, _STG // _NBUF, body, 0)
            for b in range(_NBUF):
                pltpu.make_async_copy(
                    rows_v.at[b], acc.at[dst_v.at[0]], ssems[b]).wait()

        plsc.subcore_barrier()

        pltpu.sync_copy(acc.at[pl.ds(s * _ZROWS, _ZROWS)], out_hbm.at[c, s])

    return agg


_agg64 = _make_agg(_D // 2)
_agg32 = _make_agg(_NCLS_PAD // 2)


def _mm_first(h, w, b):
    m = w.shape[1]
    m2 = m // 2

    def body(h_ref, w_ref, b_ref, o_ref):
        r = (jnp.dot(h_ref[...], w_ref[...],
                     preferred_element_type=jnp.float32)
             + b_ref[...]).astype(jnp.bfloat16)
        o_ref[0] = r[:, :m2]
        o_ref[1] = r[:, m2:]

    return pl.pallas_call(
        body,
        grid=(_N // _BN,),
        in_specs=[
            pl.BlockSpec((_BN, _D), lambda i: (i, 0)),
            pl.BlockSpec((_D, m), lambda i: (0, 0)),
            pl.BlockSpec((1, m), lambda i: (0, 0)),
        ],
        out_specs=pl.BlockSpec((2, _BN, m2), lambda i: (0, i, 0)),
        out_shape=jax.ShapeDtypeStruct((2, _N, m2), jnp.bfloat16),
    )(h, w, b.reshape(1, m))


def _mm_pair(p, w, b):
    m = w.shape[1]
    m2 = m // 2
    k2 = _D // 2

    def body(p_ref, w_ref, b_ref, o_ref):
        r = ((jnp.dot(p_ref[0], w_ref[:k2, :],
                      preferred_element_type=jnp.float32)
              + jnp.dot(p_ref[1], w_ref[k2:, :],
                        preferred_element_type=jnp.float32)
              + b_ref[...])).astype(jnp.bfloat16)
        o_ref[0] = r[:, :m2]
        o_ref[1] = r[:, m2:]

    return pl.pallas_call(
        body,
        grid=(_N // _BN,),
        in_specs=[
            pl.BlockSpec((2, _BN, k2), lambda i: (0, i, 0)),
            pl.BlockSpec((_D, m), lambda i: (0, 0)),
            pl.BlockSpec((1, m), lambda i: (0, 0)),
        ],
        out_specs=pl.BlockSpec((2, _BN, m2), lambda i: (0, i, 0)),
        out_shape=jax.ShapeDtypeStruct((2, _N, m2), jnp.bfloat16),
    )(p, w, b.reshape(1, m))


def _loss_kernel(p3, labels):
    lab3 = labels.reshape(_N // _BN, 1, _BN)
    c2 = _NCLS_PAD // 2

    def body(p_ref, lab_ref, o_ref):
        i = pl.program_id(0)
        logits = jnp.concatenate(
            [p_ref[0], p_ref[1]], axis=1).astype(jnp.float32)
        col = lax.broadcasted_iota(jnp.int32, (_BN, _NCLS_PAD), 1)
        x = jnp.where(col < _NCLS, logits, jnp.float32(-1e30))
        mx = jnp.max(x, axis=1, keepdims=True)
        lse = mx[:, 0] + jnp.log(jnp.sum(jnp.exp(x - mx), axis=1))
        lab = lab_ref[0, 0, :]
        picked = jnp.sum(
            jnp.where(col == lab[:, None], logits, 0.0), axis=1)
        part = jnp.sum(lse - picked) * jnp.float32(1.0 / _N)

        @pl.when(i == 0)
        def _():
            o_ref[...] = jnp.zeros((1, 1), jnp.float32)

        o_ref[...] += jnp.full((1, 1), 1.0, jnp.float32) * part

    out = pl.pallas_call(
        body,
        grid=(_N // _BN,),
        in_specs=[
            pl.BlockSpec((2, _BN, c2), lambda i: (0, i, 0)),
            pl.BlockSpec((1, 1, _BN), lambda i: (i, 0, 0)),
        ],
        out_specs=pl.BlockSpec((1, 1), lambda i: (0, 0)),
        out_shape=jax.ShapeDtypeStruct((1, 1), jnp.float32),
    )(p3, lab3)
    return out[0, 0]


def kernel(features, labels, edge_index, W1, b1, W2, b2, W3, b3):
    dst = edge_index[0]
    src = edge_index[1]
    pad = _EPAD - _E
    padsrc = jnp.arange(pad, dtype=jnp.int32) % _N
    paddst = _N + (jnp.arange(pad, dtype=jnp.int32) % 120)
    srcf = jnp.concatenate([src, padsrc])
    src_p = jnp.stack([srcf, srcf + _N]).reshape(_NCORES, _EROWS, _CHUNK)
    dst_p = jnp.concatenate([dst, paddst]).reshape(_EROWS, _CHUNK)
    z64 = jnp.zeros((_ZROWS, _D // 2), jnp.bfloat16)
    z32 = jnp.zeros((_ZROWS, _NCLS_PAD // 2), jnp.bfloat16)
    w3p = jnp.pad(W3, ((0, 0), (0, _NCLS_PAD - _NCLS)))
    b3p = jnp.pad(b3, (0, _NCLS_PAD - _NCLS))

    h1 = _mm_first(features, W1, b1).reshape(2 * _N, _D // 2)
    p1 = _agg64(h1, src_p, dst_p, z64)
    p1 = p1.reshape(_NCORES, _ACC_ROWS, _D // 2)
    h2 = _mm_pair(p1, W2, b2).reshape(2 * _N, _D // 2)
    p2 = _agg64(h2, src_p, dst_p, z64)
    p2 = p2.reshape(_NCORES, _ACC_ROWS, _D // 2)
    h3 = _mm_pair(p2, w3p, b3p).reshape(2 * _N, _NCLS_PAD // 2)
    p3 = _agg32(h3, src_p, dst_p, z32)
    p3 = p3.reshape(_NCORES, _ACC_ROWS, _NCLS_PAD // 2)
    return _loss_kernel(p3, labels)


# speedup vs baseline: 119.4214x; 1.0023x over previous
import functools

import jax
import jax.numpy as jnp
from jax import lax
from jax.experimental import pallas as pl
from jax.experimental.pallas import tpu as pltpu
from jax.experimental.pallas import tpu_sc as plsc

_N = 10000
_E = 320000
_D = 128
_NCLS = 40
_NCLS_PAD = 64

_NCORES = 2
_NSUB = 16
_CHUNK = 128
_CPT = 160
_STG = 40
_EPAD = _NSUB * _CPT * _CHUNK
_EROWS = _EPAD // _CHUNK
_ACC_ROWS = 10240
_ZROWS = _ACC_ROWS // _NSUB
_NBUF = 4
_BN = 2000


def _make_agg(d2):
    mesh = plsc.VectorSubcoreMesh(core_axis_name="c", subcore_axis_name="s")

    @functools.partial(
        pl.kernel,
        mesh=mesh,
        compiler_params=pltpu.CompilerParams(use_tc_tiling_on_sc=False),
        out_type=jax.ShapeDtypeStruct((_NCORES, _NSUB, _ZROWS, d2), jnp.bfloat16),
        scratch_types=[
            pltpu.VMEM((_STG, _CHUNK), jnp.int32),
            pltpu.VMEM((_STG, _CHUNK), jnp.int32),
            pltpu.VMEM((_NBUF, _CHUNK, d2), jnp.bfloat16),
            pltpu.VMEM_SHARED((_ACC_ROWS, d2), jnp.bfloat16),
            [pltpu.SemaphoreType.DMA] * _NBUF,
            [pltpu.SemaphoreType.DMA] * _NBUF,
        ],
    )
    def agg(h_hbm, src_hbm, dst_hbm, zero_hbm, out_hbm,
            src_v, dst_v, rows_v, acc, gsems, ssems):
        c = lax.axis_index("c")
        s = lax.axis_index("s")

        pltpu.sync_copy(zero_hbm, acc.at[pl.ds(s * _ZROWS, _ZROWS)])
        plsc.subcore_barrier()

        for t in range(_CPT // _STG):
            base = s * _CPT + t * _STG
            pltpu.sync_copy(src_hbm.at[c, pl.ds(base, _STG)], src_v)
            pltpu.sync_copy(dst_hbm.at[pl.ds(base, _STG)], dst_v)

            for b in range(_NBUF):
                pltpu.async_copy(h_hbm.at[src_v.at[b]], rows_v.at[b], gsems[b])

            def body(i, carry):
                for b in range(_NBUF):
                    j = i * _NBUF + b
                    pltpu.make_async_copy(
                        h_hbm.at[src_v.at[j]], rows_v.at[b], gsems[b]).wait()
                    pltpu.async_copy(
                        rows_v.at[b], acc.at[dst_v.at[j]], ssems[b], add=True)
                    nj = j + _NBUF

                    @pl.when(nj < _STG)
                    def _():
                        pltpu.make_async_copy(
                            rows_v.at[b], acc.at[dst_v.at[j]], ssems[b]).wait()
                        pltpu.async_copy(
                            h_hbm.at[src_v.at[nj]], rows_v.at[b], gsems[b])
                return carry

            lax.fori_loop(0, _STG // _NBUF, body, 0)
            for b in range(_NBUF):
                pltpu.make_async_copy(
                    rows_v.at[b], acc.at[dst_v.at[0]], ssems[b]).wait()

        plsc.subcore_barrier()

        pltpu.sync_copy(acc.at[pl.ds(s * _ZROWS, _ZROWS)], out_hbm.at[c, s])

    return agg


_agg64 = _make_agg(_D // 2)
_agg32 = _make_agg(_NCLS_PAD // 2)


def _mm_first(h, w, b):
    m = w.shape[1]
    m2 = m // 2

    def body(h_ref, w_ref, b_ref, o_ref):
        r = (jnp.dot(h_ref[...], w_ref[...],
                     preferred_element_type=jnp.float32)
             + b_ref[...]).astype(jnp.bfloat16)
        o_ref[0] = r[:, :m2]
        o_ref[1] = r[:, m2:]

    return pl.pallas_call(
        body,
        grid=(1,),
        in_specs=[
            pl.BlockSpec((_N, _D), lambda i: (0, 0)),
            pl.BlockSpec((_D, m), lambda i: (0, 0)),
            pl.BlockSpec((1, m), lambda i: (0, 0)),
        ],
        out_specs=pl.BlockSpec((2, _N, m2), lambda i: (0, 0, 0)),
        out_shape=jax.ShapeDtypeStruct((2, _N, m2), jnp.bfloat16),
    )(h, w, b.reshape(1, m))


def _mm_pair(p, w, b):
    m = w.shape[1]
    m2 = m // 2
    k2 = _D // 2

    def body(p_ref, w_ref, b_ref, o_ref):
        r = ((jnp.dot(p_ref[0], w_ref[:k2, :],
                      preferred_element_type=jnp.float32)
              + jnp.dot(p_ref[1], w_ref[k2:, :],
                        preferred_element_type=jnp.float32)
              + b_ref[...])).astype(jnp.bfloat16)
        o_ref[0] = r[:, :m2]
        o_ref[1] = r[:, m2:]

    return pl.pallas_call(
        body,
        grid=(1,),
        in_specs=[
            pl.BlockSpec((2, _N, k2), lambda i: (0, 0, 0)),
            pl.BlockSpec((_D, m), lambda i: (0, 0)),
            pl.BlockSpec((1, m), lambda i: (0, 0)),
        ],
        out_specs=pl.BlockSpec((2, _N, m2), lambda i: (0, 0, 0)),
        out_shape=jax.ShapeDtypeStruct((2, _N, m2), jnp.bfloat16),
    )(p, w, b.reshape(1, m))


def _loss_kernel(p3, labels):
    lab3 = labels.reshape(1, 1, _N)
    c2 = _NCLS_PAD // 2

    def body(p_ref, lab_ref, o_ref):
        i = pl.program_id(0)
        logits = jnp.concatenate(
            [p_ref[0], p_ref[1]], axis=1).astype(jnp.float32)
        col = lax.broadcasted_iota(jnp.int32, (_N, _NCLS_PAD), 1)
        x = jnp.where(col < _NCLS, logits, jnp.float32(-1e30))
        mx = jnp.max(x, axis=1, keepdims=True)
        lse = mx[:, 0] + jnp.log(jnp.sum(jnp.exp(x - mx), axis=1))
        lab = lab_ref[0, 0, :]
        picked = jnp.sum(
            jnp.where(col == lab[:, None], logits, 0.0), axis=1)
        part = jnp.sum(lse - picked) * jnp.float32(1.0 / _N)

        @pl.when(i == 0)
        def _():
            o_ref[...] = jnp.zeros((1, 1), jnp.float32)

        o_ref[...] += jnp.full((1, 1), 1.0, jnp.float32) * part

    out = pl.pallas_call(
        body,
        grid=(1,),
        in_specs=[
            pl.BlockSpec((2, _N, c2), lambda i: (0, 0, 0)),
            pl.BlockSpec((1, 1, _N), lambda i: (0, 0, 0)),
        ],
        out_specs=pl.BlockSpec((1, 1), lambda i: (0, 0)),
        out_shape=jax.ShapeDtypeStruct((1, 1), jnp.float32),
    )(p3, lab3)
    return out[0, 0]


def kernel(features, labels, edge_index, W1, b1, W2, b2, W3, b3):
    dst = edge_index[0]
    src = edge_index[1]
    pad = _EPAD - _E
    padsrc = jnp.arange(pad, dtype=jnp.int32) % _N
    paddst = _N + (jnp.arange(pad, dtype=jnp.int32) % 120)
    srcf = jnp.concatenate([src, padsrc])
    src_p = jnp.stack([srcf, srcf + _N]).reshape(_NCORES, _EROWS, _CHUNK)
    dst_p = jnp.concatenate([dst, paddst]).reshape(_EROWS, _CHUNK)
    z64 = jnp.zeros((_ZROWS, _D // 2), jnp.bfloat16)
    z32 = jnp.zeros((_ZROWS, _NCLS_PAD // 2), jnp.bfloat16)
    w3p = jnp.pad(W3, ((0, 0), (0, _NCLS_PAD - _NCLS)))
    b3p = jnp.pad(b3, (0, _NCLS_PAD - _NCLS))

    h1 = _mm_first(features, W1, b1).reshape(2 * _N, _D // 2)
    p1 = _agg64(h1, src_p, dst_p, z64)
    p1 = p1.reshape(_NCORES, _ACC_ROWS, _D // 2)
    h2 = _mm_pair(p1, W2, b2).reshape(2 * _N, _D // 2)
    p2 = _agg64(h2, src_p, dst_p, z64)
    p2 = p2.reshape(_NCORES, _ACC_ROWS, _D // 2)
    h3 = _mm_pair(p2, w3p, b3p).reshape(2 * _N, _NCLS_PAD // 2)
    p3 = _agg32(h3, src_p, dst_p, z32)
    p3 = p3.reshape(_NCORES, _ACC_ROWS, _NCLS_PAD // 2)
    return _loss_kernel(p3, labels)


# speedup vs baseline: 126.7107x; 1.0610x over previous
import functools

import jax
import jax.numpy as jnp
from jax import lax
from jax.experimental import pallas as pl
from jax.experimental.pallas import tpu as pltpu
from jax.experimental.pallas import tpu_sc as plsc

_N = 10000
_E = 320000
_D = 128
_NCLS = 40
_NCLS_PAD = 64

_NCORES = 2
_NSUB = 16
_CHUNK = 128
_CPT = 160
_STG = 160
_EPAD = _NSUB * _CPT * _CHUNK
_EROWS = _EPAD // _CHUNK
_ACC_ROWS = 10240
_ZROWS = _ACC_ROWS // _NSUB
_NBUF = 4
_BN = 2000


def _make_agg(d2):
    mesh = plsc.VectorSubcoreMesh(core_axis_name="c", subcore_axis_name="s")

    @functools.partial(
        pl.kernel,
        mesh=mesh,
        compiler_params=pltpu.CompilerParams(use_tc_tiling_on_sc=False),
        out_type=jax.ShapeDtypeStruct((_NCORES, _NSUB, _ZROWS, d2), jnp.bfloat16),
        scratch_types=[
            pltpu.VMEM((_STG, _CHUNK), jnp.int32),
            pltpu.VMEM((_STG, _CHUNK), jnp.int32),
            pltpu.VMEM((_NBUF, _CHUNK, d2), jnp.bfloat16),
            pltpu.VMEM_SHARED((_ACC_ROWS, d2), jnp.bfloat16),
            [pltpu.SemaphoreType.DMA] * _NBUF,
            [pltpu.SemaphoreType.DMA] * _NBUF,
        ],
    )
    def agg(h_hbm, src_hbm, dst_hbm, zero_hbm, out_hbm,
            src_v, dst_v, rows_v, acc, gsems, ssems):
        c = lax.axis_index("c")
        s = lax.axis_index("s")

        pltpu.sync_copy(zero_hbm, acc.at[pl.ds(s * _ZROWS, _ZROWS)])
        plsc.subcore_barrier()

        for t in range(_CPT // _STG):
            base = s * _CPT + t * _STG
            pltpu.sync_copy(src_hbm.at[c, pl.ds(base, _STG)], src_v)
            pltpu.sync_copy(dst_hbm.at[pl.ds(base, _STG)], dst_v)

            for b in range(_NBUF):
                pltpu.async_copy(h_hbm.at[src_v.at[b]], rows_v.at[b], gsems[b])

            def body(i, carry):
                for b in range(_NBUF):
                    j = i * _NBUF + b
                    pltpu.make_async_copy(
                        h_hbm.at[src_v.at[j]], rows_v.at[b], gsems[b]).wait()
                    pltpu.async_copy(
                        rows_v.at[b], acc.at[dst_v.at[j]], ssems[b], add=True)
                    nj = j + _NBUF

                    @pl.when(nj < _STG)
                    def _():
                        pltpu.make_async_copy(
                            rows_v.at[b], acc.at[dst_v.at[j]], ssems[b]).wait()
                        pltpu.async_copy(
                            h_hbm.at[src_v.at[nj]], rows_v.at[b], gsems[b])
                return carry

            lax.fori_loop(0, _STG // _NBUF, body, 0)
            for b in range(_NBUF):
                pltpu.make_async_copy(
                    rows_v.at[b], acc.at[dst_v.at[0]], ssems[b]).wait()

        plsc.subcore_barrier()

        pltpu.sync_copy(acc.at[pl.ds(s * _ZROWS, _ZROWS)], out_hbm.at[c, s])

    return agg


_agg64 = _make_agg(_D // 2)
_agg32 = _make_agg(_NCLS_PAD // 2)


def _mm_first(h, w, b):
    m = w.shape[1]
    m2 = m // 2

    def body(h_ref, w_ref, b_ref, o_ref):
        r = (jnp.dot(h_ref[...], w_ref[...],
                     preferred_element_type=jnp.float32)
             + b_ref[...]).astype(jnp.bfloat16)
        o_ref[0] = r[:, :m2]
        o_ref[1] = r[:, m2:]

    return pl.pallas_call(
        body,
        grid=(1,),
        in_specs=[
            pl.BlockSpec((_N, _D), lambda i: (0, 0)),
            pl.BlockSpec((_D, m), lambda i: (0, 0)),
            pl.BlockSpec((1, m), lambda i: (0, 0)),
        ],
        out_specs=pl.BlockSpec((2, _N, m2), lambda i: (0, 0, 0)),
        out_shape=jax.ShapeDtypeStruct((2, _N, m2), jnp.bfloat16),
    )(h, w, b.reshape(1, m))


def _mm_pair(p, w, b):
    m = w.shape[1]
    m2 = m // 2
    k2 = _D // 2

    def body(p_ref, w_ref, b_ref, o_ref):
        r = ((jnp.dot(p_ref[0], w_ref[:k2, :],
                      preferred_element_type=jnp.float32)
              + jnp.dot(p_ref[1], w_ref[k2:, :],
                        preferred_element_type=jnp.float32)
              + b_ref[...])).astype(jnp.bfloat16)
        o_ref[0] = r[:, :m2]
        o_ref[1] = r[:, m2:]

    return pl.pallas_call(
        body,
        grid=(1,),
        in_specs=[
            pl.BlockSpec((2, _N, k2), lambda i: (0, 0, 0)),
            pl.BlockSpec((_D, m), lambda i: (0, 0)),
            pl.BlockSpec((1, m), lambda i: (0, 0)),
        ],
        out_specs=pl.BlockSpec((2, _N, m2), lambda i: (0, 0, 0)),
        out_shape=jax.ShapeDtypeStruct((2, _N, m2), jnp.bfloat16),
    )(p, w, b.reshape(1, m))


def _loss_kernel(p3, labels):
    lab3 = labels.reshape(1, 1, _N)
    c2 = _NCLS_PAD // 2

    def body(p_ref, lab_ref, o_ref):
        i = pl.program_id(0)
        logits = jnp.concatenate(
            [p_ref[0], p_ref[1]], axis=1).astype(jnp.float32)
        col = lax.broadcasted_iota(jnp.int32, (_N, _NCLS_PAD), 1)
        x = jnp.where(col < _NCLS, logits, jnp.float32(-1e30))
        mx = jnp.max(x, axis=1, keepdims=True)
        lse = mx[:, 0] + jnp.log(jnp.sum(jnp.exp(x - mx), axis=1))
        lab = lab_ref[0, 0, :]
        picked = jnp.sum(
            jnp.where(col == lab[:, None], logits, 0.0), axis=1)
        part = jnp.sum(lse - picked) * jnp.float32(1.0 / _N)

        @pl.when(i == 0)
        def _():
            o_ref[...] = jnp.zeros((1, 1), jnp.float32)

        o_ref[...] += jnp.full((1, 1), 1.0, jnp.float32) * part

    out = pl.pallas_call(
        body,
        grid=(1,),
        in_specs=[
            pl.BlockSpec((2, _N, c2), lambda i: (0, 0, 0)),
            pl.BlockSpec((1, 1, _N), lambda i: (0, 0, 0)),
        ],
        out_specs=pl.BlockSpec((1, 1), lambda i: (0, 0)),
        out_shape=jax.ShapeDtypeStruct((1, 1), jnp.float32),
    )(p3, lab3)
    return out[0, 0]


def kernel(features, labels, edge_index, W1, b1, W2, b2, W3, b3):
    dst = edge_index[0]
    src = edge_index[1]
    pad = _EPAD - _E
    padsrc = jnp.arange(pad, dtype=jnp.int32) % _N
    paddst = _N + (jnp.arange(pad, dtype=jnp.int32) % 120)
    srcf = jnp.concatenate([src, padsrc])
    src_p = jnp.stack([srcf, srcf + _N]).reshape(_NCORES, _EROWS, _CHUNK)
    dst_p = jnp.concatenate([dst, paddst]).reshape(_EROWS, _CHUNK)
    z64 = jnp.zeros((_ZROWS, _D // 2), jnp.bfloat16)
    z32 = jnp.zeros((_ZROWS, _NCLS_PAD // 2), jnp.bfloat16)
    w3p = jnp.pad(W3, ((0, 0), (0, _NCLS_PAD - _NCLS)))
    b3p = jnp.pad(b3, (0, _NCLS_PAD - _NCLS))

    h1 = _mm_first(features, W1, b1).reshape(2 * _N, _D // 2)
    p1 = _agg64(h1, src_p, dst_p, z64)
    p1 = p1.reshape(_NCORES, _ACC_ROWS, _D // 2)
    h2 = _mm_pair(p1, W2, b2).reshape(2 * _N, _D // 2)
    p2 = _agg64(h2, src_p, dst_p, z64)
    p2 = p2.reshape(_NCORES, _ACC_ROWS, _D // 2)
    h3 = _mm_pair(p2, w3p, b3p).reshape(2 * _N, _NCLS_PAD // 2)
    p3 = _agg32(h3, src_p, dst_p, z32)
    p3 = p3.reshape(_NCORES, _ACC_ROWS, _NCLS_PAD // 2)
    return _loss_kernel(p3, labels)


# speedup vs baseline: 139.0607x; 1.0975x over previous
import functools

import jax
import jax.numpy as jnp
from jax import lax
from jax.experimental import pallas as pl
from jax.experimental.pallas import tpu as pltpu
from jax.experimental.pallas import tpu_sc as plsc

_N = 10000
_E = 320000
_D = 128
_NCLS = 40
_NCLS_PAD = 64

_NCORES = 2
_NSUB = 16
_CHUNK = 128
_CPT = 160
_STG = 160
_EPAD = _NSUB * _CPT * _CHUNK
_EROWS = _EPAD // _CHUNK
_ACC_ROWS = 10240
_ZROWS = _ACC_ROWS // _NSUB
_NBUF = 8
_BN = 2000


def _make_agg(d2):
    mesh = plsc.VectorSubcoreMesh(core_axis_name="c", subcore_axis_name="s")

    @functools.partial(
        pl.kernel,
        mesh=mesh,
        compiler_params=pltpu.CompilerParams(use_tc_tiling_on_sc=False),
        out_type=jax.ShapeDtypeStruct((_NCORES, _NSUB, _ZROWS, d2), jnp.bfloat16),
        scratch_types=[
            pltpu.VMEM((_STG, _CHUNK), jnp.int32),
            pltpu.VMEM((_STG, _CHUNK), jnp.int32),
            pltpu.VMEM((_NBUF, _CHUNK, d2), jnp.bfloat16),
            pltpu.VMEM_SHARED((_ACC_ROWS, d2), jnp.bfloat16),
            [pltpu.SemaphoreType.DMA] * _NBUF,
            [pltpu.SemaphoreType.DMA] * _NBUF,
        ],
    )
    def agg(h_hbm, src_hbm, dst_hbm, zero_hbm, out_hbm,
            src_v, dst_v, rows_v, acc, gsems, ssems):
        c = lax.axis_index("c")
        s = lax.axis_index("s")

        pltpu.sync_copy(zero_hbm, acc.at[pl.ds(s * _ZROWS, _ZROWS)])
        plsc.subcore_barrier()

        for t in range(_CPT // _STG):
            base = s * _CPT + t * _STG
            pltpu.sync_copy(src_hbm.at[c, pl.ds(base, _STG)], src_v)
            pltpu.sync_copy(dst_hbm.at[pl.ds(base, _STG)], dst_v)

            for b in range(_NBUF):
                pltpu.async_copy(h_hbm.at[src_v.at[b]], rows_v.at[b], gsems[b])

            def body(i, carry):
                for b in range(_NBUF):
                    j = i * _NBUF + b
                    pltpu.make_async_copy(
                        h_hbm.at[src_v.at[j]], rows_v.at[b], gsems[b]).wait()
                    pltpu.async_copy(
                        rows_v.at[b], acc.at[dst_v.at[j]], ssems[b], add=True)
                    nj = j + _NBUF

                    @pl.when(nj < _STG)
                    def _():
                        pltpu.make_async_copy(
                            rows_v.at[b], acc.at[dst_v.at[j]], ssems[b]).wait()
                        pltpu.async_copy(
                            h_hbm.at[src_v.at[nj]], rows_v.at[b], gsems[b])
                return carry

            lax.fori_loop(0, _STG // _NBUF, body, 0)
            for b in range(_NBUF):
                pltpu.make_async_copy(
                    rows_v.at[b], acc.at[dst_v.at[0]], ssems[b]).wait()

        plsc.subcore_barrier()

        pltpu.sync_copy(acc.at[pl.ds(s * _ZROWS, _ZROWS)], out_hbm.at[c, s])

    return agg


_agg64 = _make_agg(_D // 2)
_agg32 = _make_agg(_NCLS_PAD // 2)


def _mm_first(h, w, b):
    m = w.shape[1]
    m2 = m // 2

    def body(h_ref, w_ref, b_ref, o_ref):
        r = (jnp.dot(h_ref[...], w_ref[...],
                     preferred_element_type=jnp.float32)
             + b_ref[...]).astype(jnp.bfloat16)
        o_ref[0] = r[:, :m2]
        o_ref[1] = r[:, m2:]

    return pl.pallas_call(
        body,
        grid=(1,),
        in_specs=[
            pl.BlockSpec((_N, _D), lambda i: (0, 0)),
            pl.BlockSpec((_D, m), lambda i: (0, 0)),
            pl.BlockSpec((1, m), lambda i: (0, 0)),
        ],
        out_specs=pl.BlockSpec((2, _N, m2), lambda i: (0, 0, 0)),
        out_shape=jax.ShapeDtypeStruct((2, _N, m2), jnp.bfloat16),
    )(h, w, b.reshape(1, m))


def _mm_pair(p, w, b):
    m = w.shape[1]
    m2 = m // 2
    k2 = _D // 2

    def body(p_ref, w_ref, b_ref, o_ref):
        r = ((jnp.dot(p_ref[0], w_ref[:k2, :],
                      preferred_element_type=jnp.float32)
              + jnp.dot(p_ref[1], w_ref[k2:, :],
                        preferred_element_type=jnp.float32)
              + b_ref[...])).astype(jnp.bfloat16)
        o_ref[0] = r[:, :m2]
        o_ref[1] = r[:, m2:]

    return pl.pallas_call(
        body,
        grid=(1,),
        in_specs=[
            pl.BlockSpec((2, _N, k2), lambda i: (0, 0, 0)),
            pl.BlockSpec((_D, m), lambda i: (0, 0)),
            pl.BlockSpec((1, m), lambda i: (0, 0)),
        ],
        out_specs=pl.BlockSpec((2, _N, m2), lambda i: (0, 0, 0)),
        out_shape=jax.ShapeDtypeStruct((2, _N, m2), jnp.bfloat16),
    )(p, w, b.reshape(1, m))


def _loss_kernel(p3, labels):
    lab3 = labels.reshape(1, 1, _N)
    c2 = _NCLS_PAD // 2

    def body(p_ref, lab_ref, o_ref):
        i = pl.program_id(0)
        logits = jnp.concatenate(
            [p_ref[0], p_ref[1]], axis=1).astype(jnp.float32)
        col = lax.broadcasted_iota(jnp.int32, (_N, _NCLS_PAD), 1)
        x = jnp.where(col < _NCLS, logits, jnp.float32(-1e30))
        mx = jnp.max(x, axis=1, keepdims=True)
        lse = mx[:, 0] + jnp.log(jnp.sum(jnp.exp(x - mx), axis=1))
        lab = lab_ref[0, 0, :]
        picked = jnp.sum(
            jnp.where(col == lab[:, None], logits, 0.0), axis=1)
        part = jnp.sum(lse - picked) * jnp.float32(1.0 / _N)

        @pl.when(i == 0)
        def _():
            o_ref[...] = jnp.zeros((1, 1), jnp.float32)

        o_ref[...] += jnp.full((1, 1), 1.0, jnp.float32) * part

    out = pl.pallas_call(
        body,
        grid=(1,),
        in_specs=[
            pl.BlockSpec((2, _N, c2), lambda i: (0, 0, 0)),
            pl.BlockSpec((1, 1, _N), lambda i: (0, 0, 0)),
        ],
        out_specs=pl.BlockSpec((1, 1), lambda i: (0, 0)),
        out_shape=jax.ShapeDtypeStruct((1, 1), jnp.float32),
    )(p3, lab3)
    return out[0, 0]


def kernel(features, labels, edge_index, W1, b1, W2, b2, W3, b3):
    dst = edge_index[0]
    src = edge_index[1]
    pad = _EPAD - _E
    padsrc = jnp.arange(pad, dtype=jnp.int32) % _N
    paddst = _N + (jnp.arange(pad, dtype=jnp.int32) % 120)
    srcf = jnp.concatenate([src, padsrc])
    src_p = jnp.stack([srcf, srcf + _N]).reshape(_NCORES, _EROWS, _CHUNK)
    dst_p = jnp.concatenate([dst, paddst]).reshape(_EROWS, _CHUNK)
    z64 = jnp.zeros((_ZROWS, _D // 2), jnp.bfloat16)
    z32 = jnp.zeros((_ZROWS, _NCLS_PAD // 2), jnp.bfloat16)
    w3p = jnp.pad(W3, ((0, 0), (0, _NCLS_PAD - _NCLS)))
    b3p = jnp.pad(b3, (0, _NCLS_PAD - _NCLS))

    h1 = _mm_first(features, W1, b1).reshape(2 * _N, _D // 2)
    p1 = _agg64(h1, src_p, dst_p, z64)
    p1 = p1.reshape(_NCORES, _ACC_ROWS, _D // 2)
    h2 = _mm_pair(p1, W2, b2).reshape(2 * _N, _D // 2)
    p2 = _agg64(h2, src_p, dst_p, z64)
    p2 = p2.reshape(_NCORES, _ACC_ROWS, _D // 2)
    h3 = _mm_pair(p2, w3p, b3p).reshape(2 * _N, _NCLS_PAD // 2)
    p3 = _agg32(h3, src_p, dst_p, z32)
    p3 = p3.reshape(_NCORES, _ACC_ROWS, _NCLS_PAD // 2)
    return _loss_kernel(p3, labels)
